# baseline (speedup 1.0000x reference)
import jax
import jax.numpy as jnp
from jax.experimental import pallas as pl
from jax.experimental.pallas import tpu as pltpu

MAX_M = 16384
HIDDEN = 2048
BLOCK = 512
NB = MAX_M // BLOCK
DEPTH = 8


def _copy_body(split_ref, in_ref, passthru_ref, out_ref, bufs, vb, sem_r, sem_w, semb):
    m = split_ref[0]

    def straddle(i):
        return jnp.logical_and(m > i * BLOCK, m < i * BLOCK + BLOCK)

    def start_read(i):
        lo = i * BLOCK
        blk = pl.ds(lo, BLOCK)
        buf = bufs.at[i % DEPTH]
        sem = sem_r.at[i % DEPTH]

        @pl.when(m >= lo + BLOCK)
        def _():
            pltpu.make_async_copy(in_ref.at[blk], buf, sem).start()

        @pl.when(m <= lo)
        def _():
            pltpu.make_async_copy(passthru_ref.at[blk], buf, sem).start()

        @pl.when(straddle(i))
        def _():
            pltpu.make_async_copy(in_ref.at[blk], buf, sem).start()
            pltpu.make_async_copy(passthru_ref.at[blk], vb, semb).start()

    def wait_read(i):
        lo = i * BLOCK
        blk = pl.ds(lo, BLOCK)
        buf = bufs.at[i % DEPTH]
        sem = sem_r.at[i % DEPTH]
        pltpu.make_async_copy(in_ref.at[blk], buf, sem).wait()

        @pl.when(straddle(i))
        def _():
            pltpu.make_async_copy(passthru_ref.at[blk], vb, semb).wait()
            rows = jax.lax.broadcasted_iota(jnp.int32, (BLOCK, HIDDEN), 0) + lo
            buf[...] = jnp.where(rows < m, buf[...], vb[...])

    def start_write(i):
        blk = pl.ds(i * BLOCK, BLOCK)
        pltpu.make_async_copy(
            bufs.at[i % DEPTH], out_ref.at[blk], sem_w.at[i % DEPTH]
        ).start()

    def wait_write(i):
        blk = pl.ds(i * BLOCK, BLOCK)
        pltpu.make_async_copy(
            bufs.at[i % DEPTH], out_ref.at[blk], sem_w.at[i % DEPTH]
        ).wait()

    for i in range(NB):
        if i >= DEPTH:
            wait_write(i - DEPTH)
        start_read(i)
        if i >= 1:
            wait_read(i - 1)
            start_write(i - 1)
    wait_read(NB - 1)
    start_write(NB - 1)
    for i in range(max(0, NB - DEPTH), NB):
        wait_write(i)


def kernel(input, output, input_splits, output_splits, num_sm):
    del input_splits, num_sm
    return pl.pallas_call(
        _copy_body,
        out_shape=jax.ShapeDtypeStruct((MAX_M, HIDDEN), jnp.float32),
        in_specs=[
            pl.BlockSpec(memory_space=pltpu.SMEM),
            pl.BlockSpec(memory_space=pltpu.MemorySpace.HBM),
            pl.BlockSpec(memory_space=pltpu.MemorySpace.HBM),
        ],
        out_specs=pl.BlockSpec(memory_space=pltpu.MemorySpace.HBM),
        scratch_shapes=[
            pltpu.VMEM((DEPTH, BLOCK, HIDDEN), jnp.float32),
            pltpu.VMEM((BLOCK, HIDDEN), jnp.float32),
            pltpu.SemaphoreType.DMA((DEPTH,)),
            pltpu.SemaphoreType.DMA((DEPTH,)),
            pltpu.SemaphoreType.DMA,
        ],
    )(output_splits, input, output)


# speedup vs baseline: 1.0091x; 1.0091x over previous
import jax
import jax.numpy as jnp
from jax.experimental import pallas as pl
from jax.experimental.pallas import tpu as pltpu

MAX_M = 16384
HIDDEN = 2048
BLOCK = 1024
NB = MAX_M // BLOCK
DEPTH = 6


def _copy_body(split_ref, in_ref, passthru_ref, out_ref, bufs, vb, sem_r, sem_w, semb):
    m = split_ref[0]

    def straddle(i):
        return jnp.logical_and(m > i * BLOCK, m < i * BLOCK + BLOCK)

    def start_read(i):
        lo = i * BLOCK
        blk = pl.ds(lo, BLOCK)
        buf = bufs.at[i % DEPTH]
        sem = sem_r.at[i % DEPTH]

        @pl.when(m >= lo + BLOCK)
        def _():
            pltpu.make_async_copy(in_ref.at[blk], buf, sem).start()

        @pl.when(m <= lo)
        def _():
            pltpu.make_async_copy(passthru_ref.at[blk], buf, sem).start()

        @pl.when(straddle(i))
        def _():
            pltpu.make_async_copy(in_ref.at[blk], buf, sem).start()
            pltpu.make_async_copy(passthru_ref.at[blk], vb, semb).start()

    def wait_read(i):
        lo = i * BLOCK
        blk = pl.ds(lo, BLOCK)
        buf = bufs.at[i % DEPTH]
        sem = sem_r.at[i % DEPTH]
        pltpu.make_async_copy(in_ref.at[blk], buf, sem).wait()

        @pl.when(straddle(i))
        def _():
            pltpu.make_async_copy(passthru_ref.at[blk], vb, semb).wait()
            rows = jax.lax.broadcasted_iota(jnp.int32, (BLOCK, HIDDEN), 0) + lo
            buf[...] = jnp.where(rows < m, buf[...], vb[...])

    def start_write(i):
        blk = pl.ds(i * BLOCK, BLOCK)
        pltpu.make_async_copy(
            bufs.at[i % DEPTH], out_ref.at[blk], sem_w.at[i % DEPTH]
        ).start()

    def wait_write(i):
        blk = pl.ds(i * BLOCK, BLOCK)
        pltpu.make_async_copy(
            bufs.at[i % DEPTH], out_ref.at[blk], sem_w.at[i % DEPTH]
        ).wait()

    for i in range(NB):
        if i >= DEPTH:
            wait_write(i - DEPTH)
        start_read(i)
        if i >= 1:
            wait_read(i - 1)
            start_write(i - 1)
    wait_read(NB - 1)
    start_write(NB - 1)
    for i in range(max(0, NB - DEPTH), NB):
        wait_write(i)


def kernel(input, output, input_splits, output_splits, num_sm):
    del input_splits, num_sm
    return pl.pallas_call(
        _copy_body,
        out_shape=jax.ShapeDtypeStruct((MAX_M, HIDDEN), jnp.float32),
        in_specs=[
            pl.BlockSpec(memory_space=pltpu.SMEM),
            pl.BlockSpec(memory_space=pltpu.MemorySpace.HBM),
            pl.BlockSpec(memory_space=pltpu.MemorySpace.HBM),
        ],
        out_specs=pl.BlockSpec(memory_space=pltpu.MemorySpace.HBM),
        scratch_shapes=[
            pltpu.VMEM((DEPTH, BLOCK, HIDDEN), jnp.float32),
            pltpu.VMEM((BLOCK, HIDDEN), jnp.float32),
            pltpu.SemaphoreType.DMA((DEPTH,)),
            pltpu.SemaphoreType.DMA((DEPTH,)),
            pltpu.SemaphoreType.DMA,
        ],
    )(output_splits, input, output)
